# baseline (speedup 1.0000x reference)
import functools

import jax
import jax.numpy as jnp
from jax import lax
from jax.experimental import pallas as pl
from jax.experimental.pallas import tpu as pltpu
from jax.experimental.pallas import tpu_sc as plsc

E = 64
TOPK = 2
D = 768
HG = 512
HC = 512
OUT = 3
CAP = 256
N = 4096

ASG = N * TOPK
DUMP = E * CAP
XG_ROWS = (E + 1) * CAP

BG = 512
NW = 32
APW = ASG // NW
TPW = N // NW
DCH = 128
CCH = 64


def _gating_body(x_ref, wg1_ref, bg1_ref, gs_ref, gb_ref, wg2_ref, bg2_ref,
                 probs_ref, s1_ref, s2_ref, w1_ref, w2_ref, cnt_ref):
    i = pl.program_id(0)

    @pl.when(i == 0)
    def _():
        cnt_ref[...] = jnp.zeros_like(cnt_ref)

    x = x_ref[...]
    h = lax.dot_general(x, wg1_ref[...], (((1,), (0,)), ((), ()))) + bg1_ref[...]
    mu = jnp.mean(h, axis=-1, keepdims=True)
    var = jnp.mean((h - mu) * (h - mu), axis=-1, keepdims=True)
    h = (h - mu) / jnp.sqrt(var + 1e-5) * gs_ref[...] + gb_ref[...]
    h = jax.nn.gelu(h)
    logits = lax.dot_general(h, wg2_ref[...], (((1,), (0,)), ((), ()))) + bg2_ref[...]
    m = jnp.max(logits, axis=-1, keepdims=True)
    ex = jnp.exp(logits - m)
    probs = ex / jnp.sum(ex, axis=-1, keepdims=True)
    probs_ref[...] = probs

    eidx = lax.broadcasted_iota(jnp.int32, probs.shape, 1)
    m1 = jnp.max(probs, axis=-1, keepdims=True)
    i1 = jnp.min(jnp.where(probs == m1, eidx, E), axis=-1, keepdims=True)
    pm = jnp.where(eidx == i1, -jnp.inf, probs)
    m2 = jnp.max(pm, axis=-1, keepdims=True)
    i2 = jnp.min(jnp.where(pm == m2, eidx, E), axis=-1, keepdims=True)

    oh1 = (eidx == i1).astype(jnp.float32)
    oh2 = (eidx == i2).astype(jnp.float32)
    c = oh1 + oh2
    r = lax.broadcasted_iota(jnp.int32, (BG, BG), 0)
    col = lax.broadcasted_iota(jnp.int32, (BG, BG), 1)
    tril = (col < r).astype(jnp.float32)
    excl = lax.dot_general(tril, c, (((1,), (0,)), ((), ())))
    carry = cnt_ref[...]
    base = excl + carry
    pos1 = jnp.sum(base * oh1, axis=-1)
    pos2 = jnp.sum(base * oh2, axis=-1)
    cnt_ref[...] = carry + jnp.sum(c, axis=0, keepdims=True)

    i1f = i1[:, 0]
    i2f = i2[:, 0]
    p1 = m1[:, 0]
    p2 = m2[:, 0]
    v1 = pos1 < CAP
    v2 = pos2 < CAP
    pos1i = pos1.astype(jnp.int32)
    pos2i = pos2.astype(jnp.int32)
    s1_ref[...] = jnp.where(v1, i1f * CAP + pos1i, DUMP)
    s2_ref[...] = jnp.where(v2, i2f * CAP + pos2i, DUMP)
    w1_ref[...] = jnp.where(v1, p1, 0.0)
    w2_ref[...] = jnp.where(v2, p2, 0.0)


def _gating(x, Wg1, bg1, g_scale, g_bias, Wg2, bg2):
    nb = N // BG
    return pl.pallas_call(
        _gating_body,
        grid=(nb,),
        in_specs=[
            pl.BlockSpec((BG, D), lambda i: (i, 0)),
            pl.BlockSpec((D, HG), lambda i: (0, 0)),
            pl.BlockSpec((1, HG), lambda i: (0, 0)),
            pl.BlockSpec((1, HG), lambda i: (0, 0)),
            pl.BlockSpec((1, HG), lambda i: (0, 0)),
            pl.BlockSpec((HG, E), lambda i: (0, 0)),
            pl.BlockSpec((1, E), lambda i: (0, 0)),
        ],
        out_specs=[
            pl.BlockSpec((BG, E), lambda i: (i, 0)),
            pl.BlockSpec((BG,), lambda i: (i,)),
            pl.BlockSpec((BG,), lambda i: (i,)),
            pl.BlockSpec((BG,), lambda i: (i,)),
            pl.BlockSpec((BG,), lambda i: (i,)),
        ],
        out_shape=[
            jax.ShapeDtypeStruct((N, E), jnp.float32),
            jax.ShapeDtypeStruct((N,), jnp.int32),
            jax.ShapeDtypeStruct((N,), jnp.int32),
            jax.ShapeDtypeStruct((N,), jnp.float32),
            jax.ShapeDtypeStruct((N,), jnp.float32),
        ],
        scratch_shapes=[pltpu.VMEM((1, E), jnp.float32)],
    )(x, Wg1, bg1.reshape(1, HG), g_scale.reshape(1, HG),
      g_bias.reshape(1, HG), Wg2, bg2.reshape(1, E))


def _dispatch_body(nc, x_hbm, slot_hbm, tok_hbm, xg_hbm,
                   rows_v, gidx_v, sidx_v, sem):
    wid = lax.axis_index("s") * nc + lax.axis_index("c")
    base = wid * APW
    for j in range(APW // DCH):
        off = pl.multiple_of(base + j * DCH, DCH)
        pltpu.sync_copy(tok_hbm.at[pl.ds(off, DCH)], gidx_v)
        pltpu.sync_copy(slot_hbm.at[pl.ds(off, DCH)], sidx_v)
        pltpu.async_copy(x_hbm.at[gidx_v], rows_v, sem).wait()
        pltpu.async_copy(rows_v, xg_hbm.at[sidx_v], sem).wait()


def _dispatch(x, slot, tok):
    mesh = plsc.VectorSubcoreMesh(core_axis_name="c", subcore_axis_name="s")
    nc = mesh.num_cores
    return pl.kernel(
        functools.partial(_dispatch_body, nc),
        out_type=jax.ShapeDtypeStruct((XG_ROWS, D), jnp.float32),
        mesh=mesh,
        scratch_types=[
            pltpu.VMEM((DCH, D), jnp.float32),
            pltpu.VMEM((DCH,), jnp.int32),
            pltpu.VMEM((DCH,), jnp.int32),
            pltpu.SemaphoreType.DMA,
        ],
    )(x, slot, tok)


def _einsum_body(xg_ref, we_ref, be_ref, ya_ref):
    acc = lax.dot_general(
        xg_ref[...].astype(jnp.bfloat16), we_ref[0],
        (((1,), (0,)), ((), ())),
        preferred_element_type=jnp.float32)
    ya_ref[...] = acc + be_ref[0]


def _einsum(xg, Web, be):
    return pl.pallas_call(
        _einsum_body,
        grid=(E + 1,),
        in_specs=[
            pl.BlockSpec((CAP, D), lambda e: (e, 0)),
            pl.BlockSpec((1, D, D), lambda e: (jnp.minimum(e, E - 1), 0, 0)),
            pl.BlockSpec((1, 1, D), lambda e: (jnp.minimum(e, E - 1), 0, 0)),
        ],
        out_specs=pl.BlockSpec((CAP, D), lambda e: (e, 0)),
        out_shape=jax.ShapeDtypeStruct((XG_ROWS, D), jnp.float32),
    )(xg, Web, be.reshape(E, 1, D))


def _combine_body(nc, ya_hbm, slot_hbm, wgt_hbm, out_hbm,
                  rows_v, idx_v, wgt_v, acc_v, sem):
    wid = lax.axis_index("s") * nc + lax.axis_index("c")
    abase = wid * APW
    tbase = wid * TPW
    for j in range(APW // CCH):
        off = pl.multiple_of(abase + j * CCH, CCH)
        pltpu.sync_copy(slot_hbm.at[pl.ds(off, CCH)], idx_v)
        pltpu.sync_copy(wgt_hbm.at[pl.ds(off, CCH)], wgt_v)
        pltpu.async_copy(ya_hbm.at[idx_v], rows_v, sem).wait()

        def tok_loop(t, _):
            w0 = wgt_v[2 * t]
            w1 = wgt_v[2 * t + 1]

            def d_loop(d, _):
                v = (rows_v[2 * t, pl.ds(d * 16, 16)] * w0
                     + rows_v[2 * t + 1, pl.ds(d * 16, 16)] * w1)
                acc_v[t, pl.ds(d * 16, 16)] = v
                return 0

            lax.fori_loop(0, D // 16, d_loop, 0)
            return 0

        lax.fori_loop(0, CCH // 2, tok_loop, 0)
        pltpu.sync_copy(acc_v, out_hbm.at[pl.ds(tbase + j * (CCH // 2),
                                                CCH // 2)])


def _combine(ya, slot, wgt16):
    mesh = plsc.VectorSubcoreMesh(core_axis_name="c", subcore_axis_name="s")
    nc = mesh.num_cores
    return pl.kernel(
        functools.partial(_combine_body, nc),
        out_type=jax.ShapeDtypeStruct((N, D), jnp.float32),
        mesh=mesh,
        scratch_types=[
            pltpu.VMEM((CCH, D), jnp.float32),
            pltpu.VMEM((CCH,), jnp.int32),
            pltpu.VMEM((CCH, 16), jnp.float32),
            pltpu.VMEM((CCH // 2, D), jnp.float32),
            pltpu.SemaphoreType.DMA,
        ],
    )(ya, slot, wgt16)


def _classifier_body(cm_ref, wc1_ref, bc1_ref, cs_ref, cb_ref, wc2_ref,
                     bc2_ref, out_ref):
    h = lax.dot_general(cm_ref[...], wc1_ref[...], (((1,), (0,)), ((), ()))) + bc1_ref[...]
    mu = jnp.mean(h, axis=-1, keepdims=True)
    var = jnp.mean((h - mu) * (h - mu), axis=-1, keepdims=True)
    h = (h - mu) / jnp.sqrt(var + 1e-5) * cs_ref[...] + cb_ref[...]
    h = jnp.maximum(h, 0.0)
    out_ref[...] = lax.dot_general(h, wc2_ref[...], (((1,), (0,)), ((), ()))) + bc2_ref[...]


def _classifier(cm, Wc1, bc1, c_scale, c_bias, Wc2p, bc2p):
    nb = N // BG
    return pl.pallas_call(
        _classifier_body,
        grid=(nb,),
        in_specs=[
            pl.BlockSpec((BG, D), lambda i: (i, 0)),
            pl.BlockSpec((D, HC), lambda i: (0, 0)),
            pl.BlockSpec((1, HC), lambda i: (0, 0)),
            pl.BlockSpec((1, HC), lambda i: (0, 0)),
            pl.BlockSpec((1, HC), lambda i: (0, 0)),
            pl.BlockSpec((HC, 128), lambda i: (0, 0)),
            pl.BlockSpec((1, 128), lambda i: (0, 0)),
        ],
        out_specs=pl.BlockSpec((BG, 128), lambda i: (i, 0)),
        out_shape=jax.ShapeDtypeStruct((N, 128), jnp.float32),
    )(cm, Wc1, bc1.reshape(1, HC), c_scale.reshape(1, HC),
      c_bias.reshape(1, HC), Wc2p, bc2p)


def kernel(x, Wg1, bg1, g_scale, g_bias, Wg2, bg2, We, be,
           Wc1, bc1, c_scale, c_bias, Wc2, bc2):
    probs, s1, s2, w1, w2 = _gating(x, Wg1, bg1, g_scale, g_bias, Wg2, bg2)

    slot = jnp.stack([s1, s2], axis=-1).reshape(ASG)
    wgt = jnp.stack([w1, w2], axis=-1).reshape(ASG)
    wgt16 = jnp.broadcast_to(wgt[:, None], (ASG, 16))
    tok = (jnp.arange(ASG, dtype=jnp.int32) // TOPK).astype(jnp.int32)

    Web = We.astype(jnp.bfloat16)
    xg = _dispatch(x, slot, tok)
    ya = _einsum(xg, Web, be)
    combined = _combine(ya, slot, wgt16)

    Wc2p = jnp.pad(Wc2, ((0, 0), (0, 128 - OUT)))
    bc2p = jnp.pad(bc2, (0, 128 - OUT)).reshape(1, 128)
    logits = _classifier(combined, Wc1, bc1, c_scale, c_bias, Wc2p, bc2p)
    return (logits[:, :OUT], probs)


# speedup vs baseline: 1.1699x; 1.1699x over previous
import functools

import jax
import jax.numpy as jnp
from jax import lax
from jax.experimental import pallas as pl
from jax.experimental.pallas import tpu as pltpu
from jax.experimental.pallas import tpu_sc as plsc

E = 64
TOPK = 2
D = 768
HG = 512
HC = 512
OUT = 3
CAP = 256
N = 4096

ASG = N * TOPK
DUMP = E * CAP
XG_ROWS = (E + 1) * CAP

BG = 512
NW = 32
APW = ASG // NW
TPW = N // NW
DCH = 64
DCHN = APW // DCH
CCH = 64
CCHN = APW // CCH


def _gating_body(x_ref, wg1_ref, bg1_ref, gs_ref, gb_ref, wg2_ref, bg2_ref,
                 probs_ref, s1_ref, s2_ref, w1_ref, w2_ref, cnt_ref):
    i = pl.program_id(0)

    @pl.when(i == 0)
    def _():
        cnt_ref[...] = jnp.zeros_like(cnt_ref)

    x = x_ref[...]
    h = lax.dot_general(x, wg1_ref[...], (((1,), (0,)), ((), ()))) + bg1_ref[...]
    mu = jnp.mean(h, axis=-1, keepdims=True)
    var = jnp.mean((h - mu) * (h - mu), axis=-1, keepdims=True)
    h = (h - mu) / jnp.sqrt(var + 1e-5) * gs_ref[...] + gb_ref[...]
    h = jax.nn.gelu(h)
    logits = lax.dot_general(h, wg2_ref[...], (((1,), (0,)), ((), ()))) + bg2_ref[...]
    m = jnp.max(logits, axis=-1, keepdims=True)
    ex = jnp.exp(logits - m)
    probs = ex / jnp.sum(ex, axis=-1, keepdims=True)
    probs_ref[...] = probs

    eidx = lax.broadcasted_iota(jnp.int32, probs.shape, 1)
    m1 = jnp.max(probs, axis=-1, keepdims=True)
    i1 = jnp.min(jnp.where(probs == m1, eidx, E), axis=-1, keepdims=True)
    pm = jnp.where(eidx == i1, -jnp.inf, probs)
    m2 = jnp.max(pm, axis=-1, keepdims=True)
    i2 = jnp.min(jnp.where(pm == m2, eidx, E), axis=-1, keepdims=True)

    oh1 = (eidx == i1).astype(jnp.float32)
    oh2 = (eidx == i2).astype(jnp.float32)
    c = oh1 + oh2
    r = lax.broadcasted_iota(jnp.int32, (BG, BG), 0)
    col = lax.broadcasted_iota(jnp.int32, (BG, BG), 1)
    tril = (col < r).astype(jnp.float32)
    excl = lax.dot_general(tril, c, (((1,), (0,)), ((), ())))
    carry = cnt_ref[...]
    base = excl + carry
    pos1 = jnp.sum(base * oh1, axis=-1)
    pos2 = jnp.sum(base * oh2, axis=-1)
    cnt_ref[...] = carry + jnp.sum(c, axis=0, keepdims=True)

    i1f = i1[:, 0]
    i2f = i2[:, 0]
    p1 = m1[:, 0]
    p2 = m2[:, 0]
    v1 = pos1 < CAP
    v2 = pos2 < CAP
    pos1i = pos1.astype(jnp.int32)
    pos2i = pos2.astype(jnp.int32)
    s1_ref[...] = jnp.where(v1, i1f * CAP + pos1i, DUMP)
    s2_ref[...] = jnp.where(v2, i2f * CAP + pos2i, DUMP)
    w1_ref[...] = jnp.where(v1, p1, 0.0)
    w2_ref[...] = jnp.where(v2, p2, 0.0)


def _gating(x, Wg1, bg1, g_scale, g_bias, Wg2, bg2):
    nb = N // BG
    return pl.pallas_call(
        _gating_body,
        grid=(nb,),
        in_specs=[
            pl.BlockSpec((BG, D), lambda i: (i, 0)),
            pl.BlockSpec((D, HG), lambda i: (0, 0)),
            pl.BlockSpec((1, HG), lambda i: (0, 0)),
            pl.BlockSpec((1, HG), lambda i: (0, 0)),
            pl.BlockSpec((1, HG), lambda i: (0, 0)),
            pl.BlockSpec((HG, E), lambda i: (0, 0)),
            pl.BlockSpec((1, E), lambda i: (0, 0)),
        ],
        out_specs=[
            pl.BlockSpec((BG, E), lambda i: (i, 0)),
            pl.BlockSpec((BG,), lambda i: (i,)),
            pl.BlockSpec((BG,), lambda i: (i,)),
            pl.BlockSpec((BG,), lambda i: (i,)),
            pl.BlockSpec((BG,), lambda i: (i,)),
        ],
        out_shape=[
            jax.ShapeDtypeStruct((N, E), jnp.float32),
            jax.ShapeDtypeStruct((N,), jnp.int32),
            jax.ShapeDtypeStruct((N,), jnp.int32),
            jax.ShapeDtypeStruct((N,), jnp.float32),
            jax.ShapeDtypeStruct((N,), jnp.float32),
        ],
        scratch_shapes=[pltpu.VMEM((1, E), jnp.float32)],
    )(x, Wg1, bg1.reshape(1, HG), g_scale.reshape(1, HG),
      g_bias.reshape(1, HG), Wg2, bg2.reshape(1, E))


def _dispatch_body(nc, x_hbm, slot_hbm, tok_hbm, xg_hbm,
                   rows_a, rows_b, gidx_v, sidx_v,
                   sem_g0, sem_g1, sem_s0, sem_s1):
    wid = lax.axis_index("s") * nc + lax.axis_index("c")
    base = pl.multiple_of(wid * APW, APW)
    pltpu.sync_copy(tok_hbm.at[pl.ds(base, APW)], gidx_v)
    pltpu.sync_copy(slot_hbm.at[pl.ds(base, APW)], sidx_v)

    rows = (rows_a, rows_b)
    sem_g = (sem_g0, sem_g1)
    sem_s = (sem_s0, sem_s1)

    def gather(j):
        return pltpu.async_copy(
            x_hbm.at[gidx_v.at[pl.ds(j * DCH, DCH)]], rows[j % 2],
            sem_g[j % 2])

    def scatter(j):
        return pltpu.async_copy(
            rows[j % 2], xg_hbm.at[sidx_v.at[pl.ds(j * DCH, DCH)]],
            sem_s[j % 2])

    cpg = [None] * DCHN
    cps = [None] * DCHN
    cpg[0] = gather(0)
    for j in range(DCHN):
        if j + 1 < DCHN:
            if j >= 1:
                cps[j - 1].wait()
            cpg[j + 1] = gather(j + 1)
        cpg[j].wait()
        cps[j] = scatter(j)
    cps[DCHN - 2].wait()
    cps[DCHN - 1].wait()


def _dispatch(x, slot, tok):
    mesh = plsc.VectorSubcoreMesh(core_axis_name="c", subcore_axis_name="s")
    nc = mesh.num_cores
    return pl.kernel(
        functools.partial(_dispatch_body, nc),
        out_type=jax.ShapeDtypeStruct((XG_ROWS, D), jnp.float32),
        mesh=mesh,
        scratch_types=[
            pltpu.VMEM((DCH, D), jnp.float32),
            pltpu.VMEM((DCH, D), jnp.float32),
            pltpu.VMEM((APW,), jnp.int32),
            pltpu.VMEM((APW,), jnp.int32),
            pltpu.SemaphoreType.DMA,
            pltpu.SemaphoreType.DMA,
            pltpu.SemaphoreType.DMA,
            pltpu.SemaphoreType.DMA,
        ],
    )(x, slot, tok)


def _einsum_body(xg_ref, we_ref, be_ref, ya_ref):
    ya_ref[...] = lax.dot_general(
        xg_ref[...], we_ref[0], (((1,), (0,)), ((), ()))) + be_ref[0]


def _einsum(xg, We, be):
    return pl.pallas_call(
        _einsum_body,
        grid=(E + 1,),
        in_specs=[
            pl.BlockSpec((CAP, D), lambda e: (e, 0)),
            pl.BlockSpec((1, D, D), lambda e: (jnp.minimum(e, E - 1), 0, 0)),
            pl.BlockSpec((1, 1, D), lambda e: (jnp.minimum(e, E - 1), 0, 0)),
        ],
        out_specs=pl.BlockSpec((CAP, D), lambda e: (e, 0)),
        out_shape=jax.ShapeDtypeStruct((XG_ROWS, D), jnp.float32),
    )(xg, We, be.reshape(E, 1, D))


def _combine_body(nc, ya_hbm, slot_hbm, wgt_hbm, out_hbm,
                  rows_a, rows_b, idx_v, wgt_v, sem_g0, sem_g1):
    wid = lax.axis_index("s") * nc + lax.axis_index("c")
    abase = pl.multiple_of(wid * APW, APW)
    tbase = wid * TPW
    pltpu.sync_copy(slot_hbm.at[pl.ds(abase, APW)], idx_v)

    rows = (rows_a, rows_b)
    sem_g = (sem_g0, sem_g1)
    TCH = CCH // 2

    def gather(j):
        return pltpu.async_copy(
            ya_hbm.at[idx_v.at[pl.ds(j * CCH, CCH)]], rows[j % 2],
            sem_g[j % 2])

    cpg = [None] * CCHN
    cpg[0] = gather(0)
    for j in range(CCHN):
        if j + 1 < CCHN:
            cpg[j + 1] = gather(j + 1)
        pltpu.sync_copy(wgt_hbm.at[pl.ds(abase + j * CCH, CCH)], wgt_v)
        cpg[j].wait()
        rv = rows[j % 2]

        def tok_loop(t, _):
            w0 = wgt_v[2 * t]
            w1 = wgt_v[2 * t + 1]

            def d_loop(d, _):
                v = (rv[2 * t, pl.ds(d * 16, 16)] * w0
                     + rv[2 * t + 1, pl.ds(d * 16, 16)] * w1)
                rv[t, pl.ds(d * 16, 16)] = v
                return 0

            lax.fori_loop(0, D // 16, d_loop, 0)
            return 0

        lax.fori_loop(0, TCH, tok_loop, 0)
        pltpu.sync_copy(rv.at[pl.ds(0, TCH)],
                        out_hbm.at[pl.ds(tbase + j * TCH, TCH)])


def _combine(ya, slot, wgt16):
    mesh = plsc.VectorSubcoreMesh(core_axis_name="c", subcore_axis_name="s")
    nc = mesh.num_cores
    return pl.kernel(
        functools.partial(_combine_body, nc),
        out_type=jax.ShapeDtypeStruct((N, D), jnp.float32),
        mesh=mesh,
        scratch_types=[
            pltpu.VMEM((CCH, D), jnp.float32),
            pltpu.VMEM((CCH, D), jnp.float32),
            pltpu.VMEM((APW,), jnp.int32),
            pltpu.VMEM((CCH, 16), jnp.float32),
            pltpu.SemaphoreType.DMA,
            pltpu.SemaphoreType.DMA,
        ],
    )(ya, slot, wgt16)


def _classifier_body(cm_ref, wc1_ref, bc1_ref, cs_ref, cb_ref, wc2_ref,
                     bc2_ref, out_ref):
    h = lax.dot_general(cm_ref[...], wc1_ref[...], (((1,), (0,)), ((), ()))) + bc1_ref[...]
    mu = jnp.mean(h, axis=-1, keepdims=True)
    var = jnp.mean((h - mu) * (h - mu), axis=-1, keepdims=True)
    h = (h - mu) / jnp.sqrt(var + 1e-5) * cs_ref[...] + cb_ref[...]
    h = jnp.maximum(h, 0.0)
    out_ref[...] = lax.dot_general(h, wc2_ref[...], (((1,), (0,)), ((), ()))) + bc2_ref[...]


def _classifier(cm, Wc1, bc1, c_scale, c_bias, Wc2p, bc2p):
    nb = N // BG
    return pl.pallas_call(
        _classifier_body,
        grid=(nb,),
        in_specs=[
            pl.BlockSpec((BG, D), lambda i: (i, 0)),
            pl.BlockSpec((D, HC), lambda i: (0, 0)),
            pl.BlockSpec((1, HC), lambda i: (0, 0)),
            pl.BlockSpec((1, HC), lambda i: (0, 0)),
            pl.BlockSpec((1, HC), lambda i: (0, 0)),
            pl.BlockSpec((HC, 128), lambda i: (0, 0)),
            pl.BlockSpec((1, 128), lambda i: (0, 0)),
        ],
        out_specs=pl.BlockSpec((BG, 128), lambda i: (i, 0)),
        out_shape=jax.ShapeDtypeStruct((N, 128), jnp.float32),
    )(cm, Wc1, bc1.reshape(1, HC), c_scale.reshape(1, HC),
      c_bias.reshape(1, HC), Wc2p, bc2p)


def kernel(x, Wg1, bg1, g_scale, g_bias, Wg2, bg2, We, be,
           Wc1, bc1, c_scale, c_bias, Wc2, bc2):
    probs, s1, s2, w1, w2 = _gating(x, Wg1, bg1, g_scale, g_bias, Wg2, bg2)

    slot = jnp.stack([s1, s2], axis=-1).reshape(ASG)
    wgt = jnp.stack([w1, w2], axis=-1).reshape(ASG)
    wgt16 = jnp.broadcast_to(wgt[:, None], (ASG, 16))
    tok = (jnp.arange(ASG, dtype=jnp.int32) // TOPK).astype(jnp.int32)

    xg = _dispatch(x, slot, tok)
    ya = _einsum(xg, We, be)
    combined = _combine(ya, slot, wgt16)

    Wc2p = jnp.pad(Wc2, ((0, 0), (0, 128 - OUT)))
    bc2p = jnp.pad(bc2, (0, 128 - OUT)).reshape(1, 128)
    logits = _classifier(combined, Wc1, bc1, c_scale, c_bias, Wc2p, bc2p)
    return (logits[:, :OUT], probs)


# speedup vs baseline: 1.2485x; 1.0672x over previous
import functools

import jax
import jax.numpy as jnp
from jax import lax
from jax.experimental import pallas as pl
from jax.experimental.pallas import tpu as pltpu
from jax.experimental.pallas import tpu_sc as plsc

E = 64
TOPK = 2
D = 768
HG = 512
HC = 512
OUT = 3
CAP = 256
N = 4096

ASG = N * TOPK
DUMP = E * CAP
XG_ROWS = (E + 1) * CAP

BG = 512
NW = 32
APW = ASG // NW
TPW = N // NW
DCH = 64
DCHN = APW // DCH
CCH = 64
CCHN = APW // CCH


def _gating_body(x_ref, wg1_ref, bg1_ref, gs_ref, gb_ref, wg2_ref, bg2_ref,
                 probs_ref, s1_ref, s2_ref, w1_ref, w2_ref, cnt_ref):
    i = pl.program_id(0)

    @pl.when(i == 0)
    def _():
        cnt_ref[...] = jnp.zeros_like(cnt_ref)

    x = x_ref[...]
    h = lax.dot_general(x, wg1_ref[...], (((1,), (0,)), ((), ()))) + bg1_ref[...]
    mu = jnp.mean(h, axis=-1, keepdims=True)
    var = jnp.mean((h - mu) * (h - mu), axis=-1, keepdims=True)
    h = (h - mu) / jnp.sqrt(var + 1e-5) * gs_ref[...] + gb_ref[...]
    h = jax.nn.gelu(h)
    logits = lax.dot_general(h, wg2_ref[...], (((1,), (0,)), ((), ()))) + bg2_ref[...]
    m = jnp.max(logits, axis=-1, keepdims=True)
    ex = jnp.exp(logits - m)
    probs = ex / jnp.sum(ex, axis=-1, keepdims=True)
    probs_ref[...] = probs

    eidx = lax.broadcasted_iota(jnp.int32, probs.shape, 1)
    m1 = jnp.max(probs, axis=-1, keepdims=True)
    i1 = jnp.min(jnp.where(probs == m1, eidx, E), axis=-1, keepdims=True)
    pm = jnp.where(eidx == i1, -jnp.inf, probs)
    m2 = jnp.max(pm, axis=-1, keepdims=True)
    i2 = jnp.min(jnp.where(pm == m2, eidx, E), axis=-1, keepdims=True)

    oh1 = (eidx == i1).astype(jnp.float32)
    oh2 = (eidx == i2).astype(jnp.float32)
    c = oh1 + oh2
    r = lax.broadcasted_iota(jnp.int32, (BG, BG), 0)
    col = lax.broadcasted_iota(jnp.int32, (BG, BG), 1)
    tril = (col < r).astype(jnp.float32)
    excl = lax.dot_general(tril, c, (((1,), (0,)), ((), ())))
    carry = cnt_ref[...]
    base = excl + carry
    pos1 = jnp.sum(base * oh1, axis=-1)
    pos2 = jnp.sum(base * oh2, axis=-1)
    cnt_ref[...] = carry + jnp.sum(c, axis=0, keepdims=True)

    i1f = i1[:, 0]
    i2f = i2[:, 0]
    p1 = m1[:, 0]
    p2 = m2[:, 0]
    v1 = pos1 < CAP
    v2 = pos2 < CAP
    pos1i = pos1.astype(jnp.int32)
    pos2i = pos2.astype(jnp.int32)
    s1_ref[...] = jnp.where(v1, i1f * CAP + pos1i, DUMP)
    s2_ref[...] = jnp.where(v2, i2f * CAP + pos2i, DUMP)
    w1_ref[...] = jnp.where(v1, p1, 0.0)
    w2_ref[...] = jnp.where(v2, p2, 0.0)


def _gating(x, Wg1, bg1, g_scale, g_bias, Wg2, bg2):
    nb = N // BG
    return pl.pallas_call(
        _gating_body,
        grid=(nb,),
        in_specs=[
            pl.BlockSpec((BG, D), lambda i: (i, 0)),
            pl.BlockSpec((D, HG), lambda i: (0, 0)),
            pl.BlockSpec((1, HG), lambda i: (0, 0)),
            pl.BlockSpec((1, HG), lambda i: (0, 0)),
            pl.BlockSpec((1, HG), lambda i: (0, 0)),
            pl.BlockSpec((HG, E), lambda i: (0, 0)),
            pl.BlockSpec((1, E), lambda i: (0, 0)),
        ],
        out_specs=[
            pl.BlockSpec((BG, E), lambda i: (i, 0)),
            pl.BlockSpec((BG,), lambda i: (i,)),
            pl.BlockSpec((BG,), lambda i: (i,)),
            pl.BlockSpec((BG,), lambda i: (i,)),
            pl.BlockSpec((BG,), lambda i: (i,)),
        ],
        out_shape=[
            jax.ShapeDtypeStruct((N, E), jnp.float32),
            jax.ShapeDtypeStruct((N,), jnp.int32),
            jax.ShapeDtypeStruct((N,), jnp.int32),
            jax.ShapeDtypeStruct((N,), jnp.float32),
            jax.ShapeDtypeStruct((N,), jnp.float32),
        ],
        scratch_shapes=[pltpu.VMEM((1, E), jnp.float32)],
    )(x, Wg1, bg1.reshape(1, HG), g_scale.reshape(1, HG),
      g_bias.reshape(1, HG), Wg2, bg2.reshape(1, E))


TC2 = TPW // 2


def _dispatch_body(nc, x_hbm, s1_hbm, s2_hbm, xg_hbm,
                   rows_a, rows_b, i1_v, i2_v,
                   sem_r0, sem_r1, sem_a0, sem_a1, sem_b0, sem_b1):
    wid = lax.axis_index("s") * nc + lax.axis_index("c")
    tbase = pl.multiple_of(wid * TPW, TPW)
    pltpu.sync_copy(s1_hbm.at[pl.ds(tbase, TPW)], i1_v)
    pltpu.sync_copy(s2_hbm.at[pl.ds(tbase, TPW)], i2_v)

    rows = (rows_a, rows_b)
    sem_r = (sem_r0, sem_r1)
    sem_a = (sem_a0, sem_a1)
    sem_b = (sem_b0, sem_b1)

    def read(j):
        return pltpu.async_copy(
            x_hbm.at[pl.ds(tbase + j * TC2, TC2)], rows[j], sem_r[j])

    cps = []
    cpr = [read(0), None]
    for j in range(2):
        if j + 1 < 2:
            cpr[j + 1] = read(j + 1)
        cpr[j].wait()
        cps.append(pltpu.async_copy(
            rows[j], xg_hbm.at[i1_v.at[pl.ds(j * TC2, TC2)]], sem_a[j]))
        cps.append(pltpu.async_copy(
            rows[j], xg_hbm.at[i2_v.at[pl.ds(j * TC2, TC2)]], sem_b[j]))
    for cp in cps:
        cp.wait()


def _dispatch(x, s1, s2):
    mesh = plsc.VectorSubcoreMesh(core_axis_name="c", subcore_axis_name="s")
    nc = mesh.num_cores
    return pl.kernel(
        functools.partial(_dispatch_body, nc),
        out_type=jax.ShapeDtypeStruct((XG_ROWS, D), jnp.float32),
        mesh=mesh,
        scratch_types=[
            pltpu.VMEM((TC2, D), jnp.float32),
            pltpu.VMEM((TC2, D), jnp.float32),
            pltpu.VMEM((TPW,), jnp.int32),
            pltpu.VMEM((TPW,), jnp.int32),
            pltpu.SemaphoreType.DMA,
            pltpu.SemaphoreType.DMA,
            pltpu.SemaphoreType.DMA,
            pltpu.SemaphoreType.DMA,
            pltpu.SemaphoreType.DMA,
            pltpu.SemaphoreType.DMA,
        ],
    )(x, s1, s2)


def _einsum_body(xg_ref, we_ref, be_ref, ya_ref):
    ya_ref[...] = lax.dot_general(
        xg_ref[...], we_ref[0], (((1,), (0,)), ((), ()))) + be_ref[0]


def _einsum(xg, We, be):
    return pl.pallas_call(
        _einsum_body,
        grid=(E + 1,),
        in_specs=[
            pl.BlockSpec((CAP, D), lambda e: (e, 0)),
            pl.BlockSpec((1, D, D), lambda e: (jnp.minimum(e, E - 1), 0, 0)),
            pl.BlockSpec((1, 1, D), lambda e: (jnp.minimum(e, E - 1), 0, 0)),
        ],
        out_specs=pl.BlockSpec((CAP, D), lambda e: (e, 0)),
        out_shape=jax.ShapeDtypeStruct((XG_ROWS, D), jnp.float32),
    )(xg, We, be.reshape(E, 1, D))


def _combine_body(nc, ya_hbm, slot_hbm, wgt_hbm, out_hbm,
                  rows_a, rows_b, idx_v, wgt_v, sem_g0, sem_g1):
    wid = lax.axis_index("s") * nc + lax.axis_index("c")
    abase = pl.multiple_of(wid * APW, APW)
    tbase = wid * TPW
    pltpu.sync_copy(slot_hbm.at[pl.ds(abase, APW)], idx_v)

    rows = (rows_a, rows_b)
    sem_g = (sem_g0, sem_g1)
    TCH = CCH // 2

    def gather(j):
        return pltpu.async_copy(
            ya_hbm.at[idx_v.at[pl.ds(j * CCH, CCH)]], rows[j % 2],
            sem_g[j % 2])

    cpg = [None] * CCHN
    cpg[0] = gather(0)
    for j in range(CCHN):
        if j + 1 < CCHN:
            cpg[j + 1] = gather(j + 1)
        pltpu.sync_copy(wgt_hbm.at[pl.ds(abase + j * CCH, CCH)], wgt_v)
        cpg[j].wait()
        rv = rows[j % 2]

        def tok_loop(t, _):
            w0 = wgt_v[2 * t]
            w1 = wgt_v[2 * t + 1]

            def d_loop(d, _):
                v = (rv[2 * t, pl.ds(d * 16, 16)] * w0
                     + rv[2 * t + 1, pl.ds(d * 16, 16)] * w1)
                rv[t, pl.ds(d * 16, 16)] = v
                return 0

            lax.fori_loop(0, D // 16, d_loop, 0)
            return 0

        lax.fori_loop(0, TCH, tok_loop, 0)
        pltpu.sync_copy(rv.at[pl.ds(0, TCH)],
                        out_hbm.at[pl.ds(tbase + j * TCH, TCH)])


def _combine(ya, slot, wgt16):
    mesh = plsc.VectorSubcoreMesh(core_axis_name="c", subcore_axis_name="s")
    nc = mesh.num_cores
    return pl.kernel(
        functools.partial(_combine_body, nc),
        out_type=jax.ShapeDtypeStruct((N, D), jnp.float32),
        mesh=mesh,
        scratch_types=[
            pltpu.VMEM((CCH, D), jnp.float32),
            pltpu.VMEM((CCH, D), jnp.float32),
            pltpu.VMEM((APW,), jnp.int32),
            pltpu.VMEM((CCH, 16), jnp.float32),
            pltpu.SemaphoreType.DMA,
            pltpu.SemaphoreType.DMA,
        ],
    )(ya, slot, wgt16)


def _classifier_body(cm_ref, wc1_ref, bc1_ref, cs_ref, cb_ref, wc2_ref,
                     bc2_ref, out_ref):
    h = lax.dot_general(cm_ref[...], wc1_ref[...], (((1,), (0,)), ((), ()))) + bc1_ref[...]
    mu = jnp.mean(h, axis=-1, keepdims=True)
    var = jnp.mean((h - mu) * (h - mu), axis=-1, keepdims=True)
    h = (h - mu) / jnp.sqrt(var + 1e-5) * cs_ref[...] + cb_ref[...]
    h = jnp.maximum(h, 0.0)
    out_ref[...] = lax.dot_general(h, wc2_ref[...], (((1,), (0,)), ((), ()))) + bc2_ref[...]


def _classifier(cm, Wc1, bc1, c_scale, c_bias, Wc2p, bc2p):
    nb = N // BG
    return pl.pallas_call(
        _classifier_body,
        grid=(nb,),
        in_specs=[
            pl.BlockSpec((BG, D), lambda i: (i, 0)),
            pl.BlockSpec((D, HC), lambda i: (0, 0)),
            pl.BlockSpec((1, HC), lambda i: (0, 0)),
            pl.BlockSpec((1, HC), lambda i: (0, 0)),
            pl.BlockSpec((1, HC), lambda i: (0, 0)),
            pl.BlockSpec((HC, 128), lambda i: (0, 0)),
            pl.BlockSpec((1, 128), lambda i: (0, 0)),
        ],
        out_specs=pl.BlockSpec((BG, 128), lambda i: (i, 0)),
        out_shape=jax.ShapeDtypeStruct((N, 128), jnp.float32),
    )(cm, Wc1, bc1.reshape(1, HC), c_scale.reshape(1, HC),
      c_bias.reshape(1, HC), Wc2p, bc2p)


def kernel(x, Wg1, bg1, g_scale, g_bias, Wg2, bg2, We, be,
           Wc1, bc1, c_scale, c_bias, Wc2, bc2):
    probs, s1, s2, w1, w2 = _gating(x, Wg1, bg1, g_scale, g_bias, Wg2, bg2)

    slot = jnp.stack([s1, s2], axis=-1).reshape(ASG)
    wgt = jnp.stack([w1, w2], axis=-1).reshape(ASG)
    wgt16 = jnp.broadcast_to(wgt[:, None], (ASG, 16))

    xg = _dispatch(x, s1, s2)
    ya = _einsum(xg, We, be)
    combined = _combine(ya, slot, wgt16)

    Wc2p = jnp.pad(Wc2, ((0, 0), (0, 128 - OUT)))
    bc2p = jnp.pad(bc2, (0, 128 - OUT)).reshape(1, 128)
    logits = _classifier(combined, Wc1, bc1, c_scale, c_bias, Wc2p, bc2p)
    return (logits[:, :OUT], probs)


# speedup vs baseline: 1.2543x; 1.0046x over previous
import functools

import jax
import jax.numpy as jnp
from jax import lax
from jax.experimental import pallas as pl
from jax.experimental.pallas import tpu as pltpu
from jax.experimental.pallas import tpu_sc as plsc

E = 64
TOPK = 2
D = 768
HG = 512
HC = 512
OUT = 3
CAP = 256
N = 4096

ASG = N * TOPK
DUMP = E * CAP
XG_ROWS = (E + 1) * CAP

BG = 512
NW = 32
APW = ASG // NW
TPW = N // NW
DCH = 64
DCHN = APW // DCH
CCH = 64
CCHN = APW // CCH


def _gating_body(x_ref, wg1_ref, bg1_ref, gs_ref, gb_ref, wg2_ref, bg2_ref,
                 probs_ref, s1_ref, s2_ref, w1_ref, w2_ref, cnt_ref):
    i = pl.program_id(0)

    @pl.when(i == 0)
    def _():
        cnt_ref[...] = jnp.zeros_like(cnt_ref)

    x = x_ref[...]
    h = lax.dot_general(x, wg1_ref[...], (((1,), (0,)), ((), ()))) + bg1_ref[...]
    mu = jnp.mean(h, axis=-1, keepdims=True)
    var = jnp.mean((h - mu) * (h - mu), axis=-1, keepdims=True)
    h = (h - mu) / jnp.sqrt(var + 1e-5) * gs_ref[...] + gb_ref[...]
    h = jax.nn.gelu(h)
    logits = lax.dot_general(h, wg2_ref[...], (((1,), (0,)), ((), ()))) + bg2_ref[...]
    m = jnp.max(logits, axis=-1, keepdims=True)
    ex = jnp.exp(logits - m)
    probs = ex / jnp.sum(ex, axis=-1, keepdims=True)
    probs_ref[...] = probs

    eidx = lax.broadcasted_iota(jnp.int32, probs.shape, 1)
    m1 = jnp.max(probs, axis=-1, keepdims=True)
    i1 = jnp.min(jnp.where(probs == m1, eidx, E), axis=-1, keepdims=True)
    pm = jnp.where(eidx == i1, -jnp.inf, probs)
    m2 = jnp.max(pm, axis=-1, keepdims=True)
    i2 = jnp.min(jnp.where(pm == m2, eidx, E), axis=-1, keepdims=True)

    oh1 = (eidx == i1).astype(jnp.float32)
    oh2 = (eidx == i2).astype(jnp.float32)
    c = oh1 + oh2
    r = lax.broadcasted_iota(jnp.int32, (BG, BG), 0)
    col = lax.broadcasted_iota(jnp.int32, (BG, BG), 1)
    tril = (col < r).astype(jnp.float32)
    excl = lax.dot_general(tril, c, (((1,), (0,)), ((), ())))
    carry = cnt_ref[...]
    base = excl + carry
    pos1 = jnp.sum(base * oh1, axis=-1)
    pos2 = jnp.sum(base * oh2, axis=-1)
    cnt_ref[...] = carry + jnp.sum(c, axis=0, keepdims=True)

    i1f = i1[:, 0]
    i2f = i2[:, 0]
    p1 = m1[:, 0]
    p2 = m2[:, 0]
    v1 = pos1 < CAP
    v2 = pos2 < CAP
    pos1i = pos1.astype(jnp.int32)
    pos2i = pos2.astype(jnp.int32)
    s1_ref[...] = jnp.where(v1, i1f * CAP + pos1i, DUMP)
    s2_ref[...] = jnp.where(v2, i2f * CAP + pos2i, DUMP)
    w1_ref[...] = jnp.where(v1, p1, 0.0)
    w2_ref[...] = jnp.where(v2, p2, 0.0)


def _gating(x, Wg1, bg1, g_scale, g_bias, Wg2, bg2):
    nb = N // BG
    return pl.pallas_call(
        _gating_body,
        grid=(nb,),
        in_specs=[
            pl.BlockSpec((BG, D), lambda i: (i, 0)),
            pl.BlockSpec((D, HG), lambda i: (0, 0)),
            pl.BlockSpec((1, HG), lambda i: (0, 0)),
            pl.BlockSpec((1, HG), lambda i: (0, 0)),
            pl.BlockSpec((1, HG), lambda i: (0, 0)),
            pl.BlockSpec((HG, E), lambda i: (0, 0)),
            pl.BlockSpec((1, E), lambda i: (0, 0)),
        ],
        out_specs=[
            pl.BlockSpec((BG, E), lambda i: (i, 0)),
            pl.BlockSpec((BG,), lambda i: (i,)),
            pl.BlockSpec((BG,), lambda i: (i,)),
            pl.BlockSpec((BG,), lambda i: (i,)),
            pl.BlockSpec((BG,), lambda i: (i,)),
        ],
        out_shape=[
            jax.ShapeDtypeStruct((N, E), jnp.float32),
            jax.ShapeDtypeStruct((N,), jnp.int32),
            jax.ShapeDtypeStruct((N,), jnp.int32),
            jax.ShapeDtypeStruct((N,), jnp.float32),
            jax.ShapeDtypeStruct((N,), jnp.float32),
        ],
        scratch_shapes=[pltpu.VMEM((1, E), jnp.float32)],
    )(x, Wg1, bg1.reshape(1, HG), g_scale.reshape(1, HG),
      g_bias.reshape(1, HG), Wg2, bg2.reshape(1, E))


TC2 = TPW // 2


def _dispatch_body(nc, x_hbm, s1_hbm, s2_hbm, xg_hbm,
                   rows_a, rows_b, i1_v, i2_v,
                   sem_r0, sem_r1, sem_a0, sem_a1, sem_b0, sem_b1):
    wid = lax.axis_index("s") * nc + lax.axis_index("c")
    tbase = pl.multiple_of(wid * TPW, TPW)
    pltpu.sync_copy(s1_hbm.at[pl.ds(tbase, TPW)], i1_v)
    pltpu.sync_copy(s2_hbm.at[pl.ds(tbase, TPW)], i2_v)

    rows = (rows_a, rows_b)
    sem_r = (sem_r0, sem_r1)
    sem_a = (sem_a0, sem_a1)
    sem_b = (sem_b0, sem_b1)

    def read(j):
        return pltpu.async_copy(
            x_hbm.at[pl.ds(tbase + j * TC2, TC2)], rows[j], sem_r[j])

    cps = []
    cpr = [read(0), None]
    for j in range(2):
        if j + 1 < 2:
            cpr[j + 1] = read(j + 1)
        cpr[j].wait()
        cps.append(pltpu.async_copy(
            rows[j], xg_hbm.at[i1_v.at[pl.ds(j * TC2, TC2)]], sem_a[j]))
        cps.append(pltpu.async_copy(
            rows[j], xg_hbm.at[i2_v.at[pl.ds(j * TC2, TC2)]], sem_b[j]))
    for cp in cps:
        cp.wait()


def _dispatch(x, s1, s2):
    mesh = plsc.VectorSubcoreMesh(core_axis_name="c", subcore_axis_name="s")
    nc = mesh.num_cores
    return pl.kernel(
        functools.partial(_dispatch_body, nc),
        out_type=jax.ShapeDtypeStruct((XG_ROWS, D), jnp.float32),
        mesh=mesh,
        scratch_types=[
            pltpu.VMEM((TC2, D), jnp.float32),
            pltpu.VMEM((TC2, D), jnp.float32),
            pltpu.VMEM((TPW,), jnp.int32),
            pltpu.VMEM((TPW,), jnp.int32),
            pltpu.SemaphoreType.DMA,
            pltpu.SemaphoreType.DMA,
            pltpu.SemaphoreType.DMA,
            pltpu.SemaphoreType.DMA,
            pltpu.SemaphoreType.DMA,
            pltpu.SemaphoreType.DMA,
        ],
    )(x, s1, s2)


def _einsum_body(xg_ref, we_ref, be_ref, ya_ref):
    ya_ref[...] = lax.dot_general(
        xg_ref[...], we_ref[0], (((1,), (0,)), ((), ()))) + be_ref[0]


def _einsum(xg, We, be):
    return pl.pallas_call(
        _einsum_body,
        grid=(E + 1,),
        in_specs=[
            pl.BlockSpec((CAP, D), lambda e: (e, 0)),
            pl.BlockSpec((1, D, D), lambda e: (jnp.minimum(e, E - 1), 0, 0)),
            pl.BlockSpec((1, 1, D), lambda e: (jnp.minimum(e, E - 1), 0, 0)),
        ],
        out_specs=pl.BlockSpec((CAP, D), lambda e: (e, 0)),
        out_shape=jax.ShapeDtypeStruct((XG_ROWS, D), jnp.float32),
    )(xg, We, be.reshape(E, 1, D))


def _combine_body(nc, ya_hbm, slot_hbm, wgt_hbm, out_hbm,
                  rows_a, rows_b, idx_v, wgt_v,
                  sem_g0, sem_g1, sem_s0, sem_s1):
    wid = lax.axis_index("s") * nc + lax.axis_index("c")
    abase = pl.multiple_of(wid * APW, APW)
    tbase = wid * TPW
    pltpu.sync_copy(slot_hbm.at[pl.ds(abase, APW)], idx_v)

    rows = (rows_a, rows_b)
    sem_g = (sem_g0, sem_g1)
    sem_s = (sem_s0, sem_s1)
    TCH = CCH // 2

    def gather(j):
        return pltpu.async_copy(
            ya_hbm.at[idx_v.at[pl.ds(j * CCH, CCH)]], rows[j % 2],
            sem_g[j % 2])

    cpg = [None] * CCHN
    cst = [None] * CCHN
    cpg[0] = gather(0)
    for j in range(CCHN):
        if j + 1 < CCHN:
            if j >= 1:
                cst[j - 1].wait()
            cpg[j + 1] = gather(j + 1)
        pltpu.sync_copy(wgt_hbm.at[pl.ds(abase + j * CCH, CCH)], wgt_v)
        cpg[j].wait()
        rv = rows[j % 2]

        def tok_loop(t, _):
            w0 = wgt_v[2 * t]
            w1 = wgt_v[2 * t + 1]

            def d_loop(d, _):
                v = (rv[2 * t, pl.ds(d * 16, 16)] * w0
                     + rv[2 * t + 1, pl.ds(d * 16, 16)] * w1)
                rv[t, pl.ds(d * 16, 16)] = v
                return 0

            lax.fori_loop(0, D // 16, d_loop, 0)
            return 0

        lax.fori_loop(0, TCH, tok_loop, 0)
        cst[j] = pltpu.async_copy(
            rv.at[pl.ds(0, TCH)],
            out_hbm.at[pl.ds(tbase + j * TCH, TCH)], sem_s[j % 2])
    cst[CCHN - 2].wait()
    cst[CCHN - 1].wait()


def _combine(ya, slot, wgt16):
    mesh = plsc.VectorSubcoreMesh(core_axis_name="c", subcore_axis_name="s")
    nc = mesh.num_cores
    return pl.kernel(
        functools.partial(_combine_body, nc),
        out_type=jax.ShapeDtypeStruct((N, D), jnp.float32),
        mesh=mesh,
        scratch_types=[
            pltpu.VMEM((CCH, D), jnp.float32),
            pltpu.VMEM((CCH, D), jnp.float32),
            pltpu.VMEM((APW,), jnp.int32),
            pltpu.VMEM((CCH, 16), jnp.float32),
            pltpu.SemaphoreType.DMA,
            pltpu.SemaphoreType.DMA,
            pltpu.SemaphoreType.DMA,
            pltpu.SemaphoreType.DMA,
        ],
    )(ya, slot, wgt16)


def _classifier_body(cm_ref, wc1_ref, bc1_ref, cs_ref, cb_ref, wc2_ref,
                     bc2_ref, out_ref):
    h = lax.dot_general(cm_ref[...], wc1_ref[...], (((1,), (0,)), ((), ()))) + bc1_ref[...]
    mu = jnp.mean(h, axis=-1, keepdims=True)
    var = jnp.mean((h - mu) * (h - mu), axis=-1, keepdims=True)
    h = (h - mu) / jnp.sqrt(var + 1e-5) * cs_ref[...] + cb_ref[...]
    h = jnp.maximum(h, 0.0)
    out_ref[...] = lax.dot_general(h, wc2_ref[...], (((1,), (0,)), ((), ()))) + bc2_ref[...]


def _classifier(cm, Wc1, bc1, c_scale, c_bias, Wc2p, bc2p):
    nb = N // BG
    return pl.pallas_call(
        _classifier_body,
        grid=(nb,),
        in_specs=[
            pl.BlockSpec((BG, D), lambda i: (i, 0)),
            pl.BlockSpec((D, HC), lambda i: (0, 0)),
            pl.BlockSpec((1, HC), lambda i: (0, 0)),
            pl.BlockSpec((1, HC), lambda i: (0, 0)),
            pl.BlockSpec((1, HC), lambda i: (0, 0)),
            pl.BlockSpec((HC, 128), lambda i: (0, 0)),
            pl.BlockSpec((1, 128), lambda i: (0, 0)),
        ],
        out_specs=pl.BlockSpec((BG, 128), lambda i: (i, 0)),
        out_shape=jax.ShapeDtypeStruct((N, 128), jnp.float32),
    )(cm, Wc1, bc1.reshape(1, HC), c_scale.reshape(1, HC),
      c_bias.reshape(1, HC), Wc2p, bc2p)


def kernel(x, Wg1, bg1, g_scale, g_bias, Wg2, bg2, We, be,
           Wc1, bc1, c_scale, c_bias, Wc2, bc2):
    probs, s1, s2, w1, w2 = _gating(x, Wg1, bg1, g_scale, g_bias, Wg2, bg2)

    slot = jnp.stack([s1, s2], axis=-1).reshape(ASG)
    wgt = jnp.stack([w1, w2], axis=-1).reshape(ASG)
    wgt16 = jnp.broadcast_to(wgt[:, None], (ASG, 16))

    xg = _dispatch(x, s1, s2)
    ya = _einsum(xg, We, be)
    combined = _combine(ya, slot, wgt16)

    Wc2p = jnp.pad(Wc2, ((0, 0), (0, 128 - OUT)))
    bc2p = jnp.pad(bc2, (0, 128 - OUT)).reshape(1, 128)
    logits = _classifier(combined, Wc1, bc1, c_scale, c_bias, Wc2p, bc2p)
    return (logits[:, :OUT], probs)


# speedup vs baseline: 1.3319x; 1.0619x over previous
import functools

import jax
import jax.numpy as jnp
from jax import lax
from jax.experimental import pallas as pl
from jax.experimental.pallas import tpu as pltpu
from jax.experimental.pallas import tpu_sc as plsc

E = 64
TOPK = 2
D = 768
HG = 512
HC = 512
OUT = 3
CAP = 256
N = 4096

ASG = N * TOPK
DUMP = E * CAP
EPB = 2
EB = E // EPB + 1
XG_ROWS = EB * EPB * CAP

BG = 512
NW = 32
APW = ASG // NW
TPW = N // NW
DCH = 64
DCHN = APW // DCH
CCH = 64
CCHN = APW // CCH


def _gating_body(x_ref, wg1_ref, bg1_ref, gs_ref, gb_ref, wg2_ref, bg2_ref,
                 probs_ref, s1_ref, s2_ref, w1_ref, w2_ref, cnt_ref):
    i = pl.program_id(0)

    @pl.when(i == 0)
    def _():
        cnt_ref[...] = jnp.zeros_like(cnt_ref)

    x = x_ref[...]
    h = lax.dot_general(x, wg1_ref[...], (((1,), (0,)), ((), ()))) + bg1_ref[...]
    mu = jnp.mean(h, axis=-1, keepdims=True)
    var = jnp.mean((h - mu) * (h - mu), axis=-1, keepdims=True)
    h = (h - mu) / jnp.sqrt(var + 1e-5) * gs_ref[...] + gb_ref[...]
    h = jax.nn.gelu(h)
    logits = lax.dot_general(h, wg2_ref[...], (((1,), (0,)), ((), ()))) + bg2_ref[...]
    m = jnp.max(logits, axis=-1, keepdims=True)
    ex = jnp.exp(logits - m)
    probs = ex / jnp.sum(ex, axis=-1, keepdims=True)
    probs_ref[...] = probs

    eidx = lax.broadcasted_iota(jnp.int32, probs.shape, 1)
    m1 = jnp.max(probs, axis=-1, keepdims=True)
    i1 = jnp.min(jnp.where(probs == m1, eidx, E), axis=-1, keepdims=True)
    pm = jnp.where(eidx == i1, -jnp.inf, probs)
    m2 = jnp.max(pm, axis=-1, keepdims=True)
    i2 = jnp.min(jnp.where(pm == m2, eidx, E), axis=-1, keepdims=True)

    oh1 = (eidx == i1).astype(jnp.float32)
    oh2 = (eidx == i2).astype(jnp.float32)
    c = oh1 + oh2
    r = lax.broadcasted_iota(jnp.int32, (BG, BG), 0)
    col = lax.broadcasted_iota(jnp.int32, (BG, BG), 1)
    tril = (col < r).astype(jnp.float32)
    excl = lax.dot_general(tril, c, (((1,), (0,)), ((), ())))
    carry = cnt_ref[...]
    base = excl + carry
    pos1 = jnp.sum(base * oh1, axis=-1)
    pos2 = jnp.sum(base * oh2, axis=-1)
    cnt_ref[...] = carry + jnp.sum(c, axis=0, keepdims=True)

    i1f = i1[:, 0]
    i2f = i2[:, 0]
    p1 = m1[:, 0]
    p2 = m2[:, 0]
    v1 = pos1 < CAP
    v2 = pos2 < CAP
    pos1i = pos1.astype(jnp.int32)
    pos2i = pos2.astype(jnp.int32)
    s1_ref[...] = jnp.where(v1, i1f * CAP + pos1i, DUMP)
    s2_ref[...] = jnp.where(v2, i2f * CAP + pos2i, DUMP)
    w1_ref[...] = jnp.where(v1, p1, 0.0)
    w2_ref[...] = jnp.where(v2, p2, 0.0)


def _gating(x, Wg1, bg1, g_scale, g_bias, Wg2, bg2):
    nb = N // BG
    return pl.pallas_call(
        _gating_body,
        grid=(nb,),
        in_specs=[
            pl.BlockSpec((BG, D), lambda i: (i, 0)),
            pl.BlockSpec((D, HG), lambda i: (0, 0)),
            pl.BlockSpec((1, HG), lambda i: (0, 0)),
            pl.BlockSpec((1, HG), lambda i: (0, 0)),
            pl.BlockSpec((1, HG), lambda i: (0, 0)),
            pl.BlockSpec((HG, E), lambda i: (0, 0)),
            pl.BlockSpec((1, E), lambda i: (0, 0)),
        ],
        out_specs=[
            pl.BlockSpec((BG, E), lambda i: (i, 0)),
            pl.BlockSpec((BG,), lambda i: (i,)),
            pl.BlockSpec((BG,), lambda i: (i,)),
            pl.BlockSpec((BG,), lambda i: (i,)),
            pl.BlockSpec((BG,), lambda i: (i,)),
        ],
        out_shape=[
            jax.ShapeDtypeStruct((N, E), jnp.float32),
            jax.ShapeDtypeStruct((N,), jnp.int32),
            jax.ShapeDtypeStruct((N,), jnp.int32),
            jax.ShapeDtypeStruct((N,), jnp.float32),
            jax.ShapeDtypeStruct((N,), jnp.float32),
        ],
        scratch_shapes=[pltpu.VMEM((1, E), jnp.float32)],
    )(x, Wg1, bg1.reshape(1, HG), g_scale.reshape(1, HG),
      g_bias.reshape(1, HG), Wg2, bg2.reshape(1, E))


TC2 = TPW // 2


def _dispatch_body(nc, x_hbm, s1_hbm, s2_hbm, xg_hbm,
                   rows_a, rows_b, i1_v, i2_v,
                   sem_r0, sem_r1, sem_a0, sem_a1, sem_b0, sem_b1):
    wid = lax.axis_index("s") * nc + lax.axis_index("c")
    tbase = pl.multiple_of(wid * TPW, TPW)
    pltpu.sync_copy(s1_hbm.at[pl.ds(tbase, TPW)], i1_v)
    pltpu.sync_copy(s2_hbm.at[pl.ds(tbase, TPW)], i2_v)

    rows = (rows_a, rows_b)
    sem_r = (sem_r0, sem_r1)
    sem_a = (sem_a0, sem_a1)
    sem_b = (sem_b0, sem_b1)

    def read(j):
        return pltpu.async_copy(
            x_hbm.at[pl.ds(tbase + j * TC2, TC2)], rows[j], sem_r[j])

    cps = []
    cpr = [read(0), None]
    for j in range(2):
        if j + 1 < 2:
            cpr[j + 1] = read(j + 1)
        cpr[j].wait()
        cps.append(pltpu.async_copy(
            rows[j], xg_hbm.at[i1_v.at[pl.ds(j * TC2, TC2)]], sem_a[j]))
        cps.append(pltpu.async_copy(
            rows[j], xg_hbm.at[i2_v.at[pl.ds(j * TC2, TC2)]], sem_b[j]))
    for cp in cps:
        cp.wait()


def _dispatch(x, s1, s2):
    mesh = plsc.VectorSubcoreMesh(core_axis_name="c", subcore_axis_name="s")
    nc = mesh.num_cores
    return pl.kernel(
        functools.partial(_dispatch_body, nc),
        out_type=jax.ShapeDtypeStruct((XG_ROWS, D), jnp.float32),
        mesh=mesh,
        scratch_types=[
            pltpu.VMEM((TC2, D), jnp.float32),
            pltpu.VMEM((TC2, D), jnp.float32),
            pltpu.VMEM((TPW,), jnp.int32),
            pltpu.VMEM((TPW,), jnp.int32),
            pltpu.SemaphoreType.DMA,
            pltpu.SemaphoreType.DMA,
            pltpu.SemaphoreType.DMA,
            pltpu.SemaphoreType.DMA,
            pltpu.SemaphoreType.DMA,
            pltpu.SemaphoreType.DMA,
        ],
    )(x, s1, s2)


def _einsum_body(xg_ref, we_ref, be_ref, ya_ref):
    for k in range(EPB):
        ya_ref[k * CAP:(k + 1) * CAP, :] = lax.dot_general(
            xg_ref[k * CAP:(k + 1) * CAP, :], we_ref[k],
            (((1,), (0,)), ((), ()))) + be_ref[k]


def _einsum(xg, We, be):
    return pl.pallas_call(
        _einsum_body,
        grid=(EB,),
        in_specs=[
            pl.BlockSpec((EPB * CAP, D), lambda e: (e, 0)),
            pl.BlockSpec((EPB, D, D),
                         lambda e: (jnp.minimum(e, E // EPB - 1), 0, 0)),
            pl.BlockSpec((EPB, 1, D),
                         lambda e: (jnp.minimum(e, E // EPB - 1), 0, 0)),
        ],
        out_specs=pl.BlockSpec((EPB * CAP, D), lambda e: (e, 0)),
        out_shape=jax.ShapeDtypeStruct((XG_ROWS, D), jnp.float32),
    )(xg, We, be.reshape(E, 1, D))


def _combine_body(nc, ya_hbm, slot_hbm, wgt_hbm, out_hbm,
                  rows_a, rows_b, idx_v, wgt_v,
                  sem_g0, sem_g1, sem_s0, sem_s1):
    wid = lax.axis_index("s") * nc + lax.axis_index("c")
    abase = pl.multiple_of(wid * APW, APW)
    tbase = wid * TPW
    pltpu.sync_copy(slot_hbm.at[pl.ds(abase, APW)], idx_v)

    rows = (rows_a, rows_b)
    sem_g = (sem_g0, sem_g1)
    sem_s = (sem_s0, sem_s1)
    TCH = CCH // 2

    def gather(j):
        return pltpu.async_copy(
            ya_hbm.at[idx_v.at[pl.ds(j * CCH, CCH)]], rows[j % 2],
            sem_g[j % 2])

    cpg = [None] * CCHN
    cst = [None] * CCHN
    cpg[0] = gather(0)
    for j in range(CCHN):
        if j + 1 < CCHN:
            if j >= 1:
                cst[j - 1].wait()
            cpg[j + 1] = gather(j + 1)
        pltpu.sync_copy(wgt_hbm.at[pl.ds(abase + j * CCH, CCH)], wgt_v)
        cpg[j].wait()
        rv = rows[j % 2]

        def tok_loop(t, _):
            w0 = wgt_v[2 * t]
            w1 = wgt_v[2 * t + 1]

            def d_loop(d, _):
                v = (rv[2 * t, pl.ds(d * 16, 16)] * w0
                     + rv[2 * t + 1, pl.ds(d * 16, 16)] * w1)
                rv[t, pl.ds(d * 16, 16)] = v
                return 0

            lax.fori_loop(0, D // 16, d_loop, 0)
            return 0

        lax.fori_loop(0, TCH, tok_loop, 0)
        cst[j] = pltpu.async_copy(
            rv.at[pl.ds(0, TCH)],
            out_hbm.at[pl.ds(tbase + j * TCH, TCH)], sem_s[j % 2])
    cst[CCHN - 2].wait()
    cst[CCHN - 1].wait()


def _combine(ya, slot, wgt16):
    mesh = plsc.VectorSubcoreMesh(core_axis_name="c", subcore_axis_name="s")
    nc = mesh.num_cores
    return pl.kernel(
        functools.partial(_combine_body, nc),
        out_type=jax.ShapeDtypeStruct((N, D), jnp.float32),
        mesh=mesh,
        scratch_types=[
            pltpu.VMEM((CCH, D), jnp.float32),
            pltpu.VMEM((CCH, D), jnp.float32),
            pltpu.VMEM((APW,), jnp.int32),
            pltpu.VMEM((CCH, 16), jnp.float32),
            pltpu.SemaphoreType.DMA,
            pltpu.SemaphoreType.DMA,
            pltpu.SemaphoreType.DMA,
            pltpu.SemaphoreType.DMA,
        ],
    )(ya, slot, wgt16)


def _classifier_body(cm_ref, wc1_ref, bc1_ref, cs_ref, cb_ref, wc2_ref,
                     bc2_ref, out_ref):
    h = lax.dot_general(cm_ref[...], wc1_ref[...], (((1,), (0,)), ((), ()))) + bc1_ref[...]
    mu = jnp.mean(h, axis=-1, keepdims=True)
    var = jnp.mean((h - mu) * (h - mu), axis=-1, keepdims=True)
    h = (h - mu) / jnp.sqrt(var + 1e-5) * cs_ref[...] + cb_ref[...]
    h = jnp.maximum(h, 0.0)
    out_ref[...] = lax.dot_general(h, wc2_ref[...], (((1,), (0,)), ((), ()))) + bc2_ref[...]


def _classifier(cm, Wc1, bc1, c_scale, c_bias, Wc2p, bc2p):
    nb = N // BG
    return pl.pallas_call(
        _classifier_body,
        grid=(nb,),
        in_specs=[
            pl.BlockSpec((BG, D), lambda i: (i, 0)),
            pl.BlockSpec((D, HC), lambda i: (0, 0)),
            pl.BlockSpec((1, HC), lambda i: (0, 0)),
            pl.BlockSpec((1, HC), lambda i: (0, 0)),
            pl.BlockSpec((1, HC), lambda i: (0, 0)),
            pl.BlockSpec((HC, 128), lambda i: (0, 0)),
            pl.BlockSpec((1, 128), lambda i: (0, 0)),
        ],
        out_specs=pl.BlockSpec((BG, 128), lambda i: (i, 0)),
        out_shape=jax.ShapeDtypeStruct((N, 128), jnp.float32),
    )(cm, Wc1, bc1.reshape(1, HC), c_scale.reshape(1, HC),
      c_bias.reshape(1, HC), Wc2p, bc2p)


def kernel(x, Wg1, bg1, g_scale, g_bias, Wg2, bg2, We, be,
           Wc1, bc1, c_scale, c_bias, Wc2, bc2):
    probs, s1, s2, w1, w2 = _gating(x, Wg1, bg1, g_scale, g_bias, Wg2, bg2)

    slot = jnp.stack([s1, s2], axis=-1).reshape(ASG)
    wgt = jnp.stack([w1, w2], axis=-1).reshape(ASG)
    wgt16 = jnp.broadcast_to(wgt[:, None], (ASG, 16))

    xg = _dispatch(x, s1, s2)
    ya = _einsum(xg, We, be)
    combined = _combine(ya, slot, wgt16)

    Wc2p = jnp.pad(Wc2, ((0, 0), (0, 128 - OUT)))
    bc2p = jnp.pad(bc2, (0, 128 - OUT)).reshape(1, 128)
    logits = _classifier(combined, Wc1, bc1, c_scale, c_bias, Wc2p, bc2p)
    return (logits[:, :OUT], probs)


# speedup vs baseline: 1.3380x; 1.0046x over previous
import functools

import jax
import jax.numpy as jnp
from jax import lax
from jax.experimental import pallas as pl
from jax.experimental.pallas import tpu as pltpu
from jax.experimental.pallas import tpu_sc as plsc

E = 64
TOPK = 2
D = 768
HG = 512
HC = 512
OUT = 3
CAP = 256
N = 4096

ASG = N * TOPK
DUMP = E * CAP
EPB = 4
EB = E // EPB + 1
XG_ROWS = EB * EPB * CAP

BG = 512
NW = 32
APW = ASG // NW
TPW = N // NW
DCH = 64
DCHN = APW // DCH
CCH = 64
CCHN = APW // CCH


def _gating_body(x_ref, wg1_ref, bg1_ref, gs_ref, gb_ref, wg2_ref, bg2_ref,
                 probs_ref, s1_ref, s2_ref, w1_ref, w2_ref, cnt_ref):
    i = pl.program_id(0)

    @pl.when(i == 0)
    def _():
        cnt_ref[...] = jnp.zeros_like(cnt_ref)

    x = x_ref[...]
    h = lax.dot_general(x, wg1_ref[...], (((1,), (0,)), ((), ()))) + bg1_ref[...]
    mu = jnp.mean(h, axis=-1, keepdims=True)
    var = jnp.mean((h - mu) * (h - mu), axis=-1, keepdims=True)
    h = (h - mu) / jnp.sqrt(var + 1e-5) * gs_ref[...] + gb_ref[...]
    h = jax.nn.gelu(h)
    logits = lax.dot_general(h, wg2_ref[...], (((1,), (0,)), ((), ()))) + bg2_ref[...]
    m = jnp.max(logits, axis=-1, keepdims=True)
    ex = jnp.exp(logits - m)
    probs = ex / jnp.sum(ex, axis=-1, keepdims=True)
    probs_ref[...] = probs

    eidx = lax.broadcasted_iota(jnp.int32, probs.shape, 1)
    m1 = jnp.max(probs, axis=-1, keepdims=True)
    i1 = jnp.min(jnp.where(probs == m1, eidx, E), axis=-1, keepdims=True)
    pm = jnp.where(eidx == i1, -jnp.inf, probs)
    m2 = jnp.max(pm, axis=-1, keepdims=True)
    i2 = jnp.min(jnp.where(pm == m2, eidx, E), axis=-1, keepdims=True)

    oh1 = (eidx == i1).astype(jnp.float32)
    oh2 = (eidx == i2).astype(jnp.float32)
    c = oh1 + oh2
    r = lax.broadcasted_iota(jnp.int32, (BG, BG), 0)
    col = lax.broadcasted_iota(jnp.int32, (BG, BG), 1)
    tril = (col < r).astype(jnp.float32)
    excl = lax.dot_general(tril, c, (((1,), (0,)), ((), ())))
    carry = cnt_ref[...]
    base = excl + carry
    pos1 = jnp.sum(base * oh1, axis=-1)
    pos2 = jnp.sum(base * oh2, axis=-1)
    cnt_ref[...] = carry + jnp.sum(c, axis=0, keepdims=True)

    i1f = i1[:, 0]
    i2f = i2[:, 0]
    p1 = m1[:, 0]
    p2 = m2[:, 0]
    v1 = pos1 < CAP
    v2 = pos2 < CAP
    pos1i = pos1.astype(jnp.int32)
    pos2i = pos2.astype(jnp.int32)
    s1_ref[...] = jnp.where(v1, i1f * CAP + pos1i, DUMP)
    s2_ref[...] = jnp.where(v2, i2f * CAP + pos2i, DUMP)
    w1_ref[...] = jnp.where(v1, p1, 0.0)
    w2_ref[...] = jnp.where(v2, p2, 0.0)


def _gating(x, Wg1, bg1, g_scale, g_bias, Wg2, bg2):
    nb = N // BG
    return pl.pallas_call(
        _gating_body,
        grid=(nb,),
        in_specs=[
            pl.BlockSpec((BG, D), lambda i: (i, 0)),
            pl.BlockSpec((D, HG), lambda i: (0, 0)),
            pl.BlockSpec((1, HG), lambda i: (0, 0)),
            pl.BlockSpec((1, HG), lambda i: (0, 0)),
            pl.BlockSpec((1, HG), lambda i: (0, 0)),
            pl.BlockSpec((HG, E), lambda i: (0, 0)),
            pl.BlockSpec((1, E), lambda i: (0, 0)),
        ],
        out_specs=[
            pl.BlockSpec((BG, E), lambda i: (i, 0)),
            pl.BlockSpec((BG,), lambda i: (i,)),
            pl.BlockSpec((BG,), lambda i: (i,)),
            pl.BlockSpec((BG,), lambda i: (i,)),
            pl.BlockSpec((BG,), lambda i: (i,)),
        ],
        out_shape=[
            jax.ShapeDtypeStruct((N, E), jnp.float32),
            jax.ShapeDtypeStruct((N,), jnp.int32),
            jax.ShapeDtypeStruct((N,), jnp.int32),
            jax.ShapeDtypeStruct((N,), jnp.float32),
            jax.ShapeDtypeStruct((N,), jnp.float32),
        ],
        scratch_shapes=[pltpu.VMEM((1, E), jnp.float32)],
    )(x, Wg1, bg1.reshape(1, HG), g_scale.reshape(1, HG),
      g_bias.reshape(1, HG), Wg2, bg2.reshape(1, E))


TC2 = TPW // 2


def _dispatch_body(nc, x_hbm, s1_hbm, s2_hbm, xg_hbm,
                   rows_a, rows_b, i1_v, i2_v,
                   sem_r0, sem_r1, sem_a0, sem_a1, sem_b0, sem_b1):
    wid = lax.axis_index("s") * nc + lax.axis_index("c")
    tbase = pl.multiple_of(wid * TPW, TPW)
    pltpu.sync_copy(s1_hbm.at[pl.ds(tbase, TPW)], i1_v)
    pltpu.sync_copy(s2_hbm.at[pl.ds(tbase, TPW)], i2_v)

    rows = (rows_a, rows_b)
    sem_r = (sem_r0, sem_r1)
    sem_a = (sem_a0, sem_a1)
    sem_b = (sem_b0, sem_b1)

    def read(j):
        return pltpu.async_copy(
            x_hbm.at[pl.ds(tbase + j * TC2, TC2)], rows[j], sem_r[j])

    cps = []
    cpr = [read(0), None]
    for j in range(2):
        if j + 1 < 2:
            cpr[j + 1] = read(j + 1)
        cpr[j].wait()
        cps.append(pltpu.async_copy(
            rows[j], xg_hbm.at[i1_v.at[pl.ds(j * TC2, TC2)]], sem_a[j]))
        cps.append(pltpu.async_copy(
            rows[j], xg_hbm.at[i2_v.at[pl.ds(j * TC2, TC2)]], sem_b[j]))
    for cp in cps:
        cp.wait()


def _dispatch(x, s1, s2):
    mesh = plsc.VectorSubcoreMesh(core_axis_name="c", subcore_axis_name="s")
    nc = mesh.num_cores
    return pl.kernel(
        functools.partial(_dispatch_body, nc),
        out_type=jax.ShapeDtypeStruct((XG_ROWS, D), jnp.float32),
        mesh=mesh,
        scratch_types=[
            pltpu.VMEM((TC2, D), jnp.float32),
            pltpu.VMEM((TC2, D), jnp.float32),
            pltpu.VMEM((TPW,), jnp.int32),
            pltpu.VMEM((TPW,), jnp.int32),
            pltpu.SemaphoreType.DMA,
            pltpu.SemaphoreType.DMA,
            pltpu.SemaphoreType.DMA,
            pltpu.SemaphoreType.DMA,
            pltpu.SemaphoreType.DMA,
            pltpu.SemaphoreType.DMA,
        ],
    )(x, s1, s2)


def _einsum_body(xg_ref, we_ref, be_ref, ya_ref):
    for k in range(EPB):
        ya_ref[k * CAP:(k + 1) * CAP, :] = lax.dot_general(
            xg_ref[k * CAP:(k + 1) * CAP, :], we_ref[k],
            (((1,), (0,)), ((), ()))) + be_ref[k]


def _einsum(xg, We, be):
    return pl.pallas_call(
        _einsum_body,
        grid=(EB,),
        in_specs=[
            pl.BlockSpec((EPB * CAP, D), lambda e: (e, 0)),
            pl.BlockSpec((EPB, D, D),
                         lambda e: (jnp.minimum(e, E // EPB - 1), 0, 0)),
            pl.BlockSpec((EPB, 1, D),
                         lambda e: (jnp.minimum(e, E // EPB - 1), 0, 0)),
        ],
        out_specs=pl.BlockSpec((EPB * CAP, D), lambda e: (e, 0)),
        out_shape=jax.ShapeDtypeStruct((XG_ROWS, D), jnp.float32),
    )(xg, We, be.reshape(E, 1, D))


def _combine_body(nc, ya_hbm, slot_hbm, wgt_hbm, out_hbm,
                  rows_a, rows_b, idx_v, wgt_v,
                  sem_g0, sem_g1, sem_s0, sem_s1):
    wid = lax.axis_index("s") * nc + lax.axis_index("c")
    abase = pl.multiple_of(wid * APW, APW)
    tbase = wid * TPW
    pltpu.sync_copy(slot_hbm.at[pl.ds(abase, APW)], idx_v)

    rows = (rows_a, rows_b)
    sem_g = (sem_g0, sem_g1)
    sem_s = (sem_s0, sem_s1)
    TCH = CCH // 2

    def gather(j):
        return pltpu.async_copy(
            ya_hbm.at[idx_v.at[pl.ds(j * CCH, CCH)]], rows[j % 2],
            sem_g[j % 2])

    cpg = [None] * CCHN
    cst = [None] * CCHN
    cpg[0] = gather(0)
    for j in range(CCHN):
        if j + 1 < CCHN:
            if j >= 1:
                cst[j - 1].wait()
            cpg[j + 1] = gather(j + 1)
        pltpu.sync_copy(wgt_hbm.at[pl.ds(abase + j * CCH, CCH)], wgt_v)
        cpg[j].wait()
        rv = rows[j % 2]

        def tok_loop(t, _):
            w0 = wgt_v[2 * t]
            w1 = wgt_v[2 * t + 1]

            def d_loop(d, _):
                v = (rv[2 * t, pl.ds(d * 16, 16)] * w0
                     + rv[2 * t + 1, pl.ds(d * 16, 16)] * w1)
                rv[t, pl.ds(d * 16, 16)] = v
                return 0

            lax.fori_loop(0, D // 16, d_loop, 0)
            return 0

        lax.fori_loop(0, TCH, tok_loop, 0)
        cst[j] = pltpu.async_copy(
            rv.at[pl.ds(0, TCH)],
            out_hbm.at[pl.ds(tbase + j * TCH, TCH)], sem_s[j % 2])
    cst[CCHN - 2].wait()
    cst[CCHN - 1].wait()


def _combine(ya, slot, wgt16):
    mesh = plsc.VectorSubcoreMesh(core_axis_name="c", subcore_axis_name="s")
    nc = mesh.num_cores
    return pl.kernel(
        functools.partial(_combine_body, nc),
        out_type=jax.ShapeDtypeStruct((N, D), jnp.float32),
        mesh=mesh,
        scratch_types=[
            pltpu.VMEM((CCH, D), jnp.float32),
            pltpu.VMEM((CCH, D), jnp.float32),
            pltpu.VMEM((APW,), jnp.int32),
            pltpu.VMEM((CCH, 16), jnp.float32),
            pltpu.SemaphoreType.DMA,
            pltpu.SemaphoreType.DMA,
            pltpu.SemaphoreType.DMA,
            pltpu.SemaphoreType.DMA,
        ],
    )(ya, slot, wgt16)


def _classifier_body(cm_ref, wc1_ref, bc1_ref, cs_ref, cb_ref, wc2_ref,
                     bc2_ref, out_ref):
    h = lax.dot_general(cm_ref[...], wc1_ref[...], (((1,), (0,)), ((), ()))) + bc1_ref[...]
    mu = jnp.mean(h, axis=-1, keepdims=True)
    var = jnp.mean((h - mu) * (h - mu), axis=-1, keepdims=True)
    h = (h - mu) / jnp.sqrt(var + 1e-5) * cs_ref[...] + cb_ref[...]
    h = jnp.maximum(h, 0.0)
    out_ref[...] = lax.dot_general(h, wc2_ref[...], (((1,), (0,)), ((), ()))) + bc2_ref[...]


def _classifier(cm, Wc1, bc1, c_scale, c_bias, Wc2p, bc2p):
    nb = N // BG
    return pl.pallas_call(
        _classifier_body,
        grid=(nb,),
        in_specs=[
            pl.BlockSpec((BG, D), lambda i: (i, 0)),
            pl.BlockSpec((D, HC), lambda i: (0, 0)),
            pl.BlockSpec((1, HC), lambda i: (0, 0)),
            pl.BlockSpec((1, HC), lambda i: (0, 0)),
            pl.BlockSpec((1, HC), lambda i: (0, 0)),
            pl.BlockSpec((HC, 128), lambda i: (0, 0)),
            pl.BlockSpec((1, 128), lambda i: (0, 0)),
        ],
        out_specs=pl.BlockSpec((BG, 128), lambda i: (i, 0)),
        out_shape=jax.ShapeDtypeStruct((N, 128), jnp.float32),
    )(cm, Wc1, bc1.reshape(1, HC), c_scale.reshape(1, HC),
      c_bias.reshape(1, HC), Wc2p, bc2p)


def kernel(x, Wg1, bg1, g_scale, g_bias, Wg2, bg2, We, be,
           Wc1, bc1, c_scale, c_bias, Wc2, bc2):
    probs, s1, s2, w1, w2 = _gating(x, Wg1, bg1, g_scale, g_bias, Wg2, bg2)

    slot = jnp.stack([s1, s2], axis=-1).reshape(ASG)
    wgt = jnp.stack([w1, w2], axis=-1).reshape(ASG)
    wgt16 = jnp.broadcast_to(wgt[:, None], (ASG, 16))

    xg = _dispatch(x, s1, s2)
    ya = _einsum(xg, We, be)
    combined = _combine(ya, slot, wgt16)

    Wc2p = jnp.pad(Wc2, ((0, 0), (0, 128 - OUT)))
    bc2p = jnp.pad(bc2, (0, 128 - OUT)).reshape(1, 128)
    logits = _classifier(combined, Wc1, bc1, c_scale, c_bias, Wc2p, bc2p)
    return (logits[:, :OUT], probs)
